# baseline (speedup 1.0000x reference)
import functools

import jax
import jax.numpy as jnp
from jax import lax
from jax.experimental import pallas as pl
from jax.experimental.pallas import tpu as pltpu
from jax.experimental.pallas import tpu_sc as plsc

B = 16384
DIM = 64
NW = 32
ROWS_PER_W = B // NW
GS = 4
GROUPS = ROWS_PER_W // GS
PAD = 16


def _body(mT_hbm, g_hbm, midx_hbm, gidx_hbm, perm_hbm, wb_hbm, out_hbm,
          midx_v, gidx_v, perm_v, win0_v, win1_v, gr0_v, gr1_v, res_v,
          tmp_v, wb_v, sem0, sem1):
    wid = lax.axis_index("s") * 2 + lax.axis_index("c")
    base = wid * ROWS_PER_W

    pltpu.sync_copy(wb_hbm, wb_v)
    for c in range(ROWS_PER_W // 128):
        pltpu.sync_copy(perm_hbm.at[pl.ds(base + c * 128, 128)],
                        perm_v.at[c])
    for c in range(ROWS_PER_W // 128):
        pltpu.async_copy(midx_hbm.at[perm_v.at[c]],
                         midx_v.at[pl.ds(c * 128, 128)], sem0)
        pltpu.async_copy(gidx_hbm.at[perm_v.at[c]],
                         gidx_v.at[pl.ds(c * 128, 128)], sem1)
    for c in range(ROWS_PER_W // 128):
        pltpu.make_async_copy(midx_hbm.at[pl.ds(0, 128)],
                              midx_v.at[pl.ds(c * 128, 128)], sem0).wait()
        pltpu.make_async_copy(gidx_hbm.at[pl.ds(0, 128)],
                              gidx_v.at[pl.ds(c * 128, 128)], sem1).wait()

    w = wb_v[0, :]
    bb = wb_v[1, :]
    iota = lax.iota(jnp.int32, 16)
    samp16 = iota & 3
    jq = iota >> 2
    fold8 = iota ^ 8
    fold4 = iota ^ 4
    mask4 = iota < 4
    zero16 = jnp.zeros((16,), jnp.int32)
    one16 = jnp.ones((16,), jnp.int32)
    two16 = jnp.full((16,), 2, jnp.int32)

    def windows(g):
        mv = midx_v[pl.ds(g * GS, 16)]
        ws = [mv[k] >> 7 for k in range(GS)]
        news = [True] + [ws[k] != ws[k - 1] for k in range(1, GS)]
        return ws, news

    def fire(g, win_v, gr_v, sem):
        mv = midx_v[pl.ds(g * GS, 16)]
        gv = gidx_v[pl.ds(g * GS, 16)]
        ws, news = windows(g)
        for k in range(GS):
            woff = pl.multiple_of(ws[k] * 128, 128)
            if k == 0:
                pltpu.async_copy(mT_hbm.at[:, pl.ds(woff, 128)],
                                 win_v.at[k], sem)
            else:
                @pl.when(news[k])
                def _(woff=woff, k=k):
                    pltpu.async_copy(mT_hbm.at[:, pl.ds(woff, 128)],
                                     win_v.at[k], sem)
            pltpu.async_copy(g_hbm.at[pl.ds(gv[k], 1)],
                             gr_v.at[pl.ds(k, 1)], sem)

    def drain(g, win_v, gr_v, sem):
        _, news = windows(g)
        for k in range(GS):
            if k == 0:
                pltpu.make_async_copy(mT_hbm.at[:, pl.ds(0, 128)],
                                      win_v.at[k], sem).wait()
            else:
                @pl.when(news[k])
                def _(k=k):
                    pltpu.make_async_copy(mT_hbm.at[:, pl.ds(0, 128)],
                                          win_v.at[k], sem).wait()
        pltpu.make_async_copy(g_hbm.at[pl.ds(0, GS)], gr_v, sem).wait()

    def compute(g, win_v, gr_v):
        goff = g * GS
        ws, news = windows(g)
        slots = [jnp.int32(0)]
        for k in range(1, GS):
            slots.append(jnp.where(news[k], jnp.int32(k), slots[k - 1]))
        slotvec = jnp.where(samp16 == 0, slots[0],
                            jnp.where(samp16 == 1, slots[1],
                                      jnp.where(samp16 == 2, slots[2],
                                                slots[3])))
        lanevec = plsc.load_gather(midx_v, [goff + samp16]) & 127
        zero = jnp.zeros((16,), jnp.float32)
        dot = zero
        nm2 = zero
        ng2 = zero
        for jb in range(0, DIM, 4):
            jv = jq + jb
            mj = plsc.load_gather(win_v, [slotvec, jv, lanevec])
            gj = plsc.load_gather(gr_v, [samp16, jv])
            dot = dot + mj * gj
            nm2 = nm2 + mj * mj
            ng2 = ng2 + gj * gj
        for fold in (fold8, fold4):
            tmp_v[0, :] = dot
            tmp_v[1, :] = nm2
            tmp_v[2, :] = ng2
            dot = dot + plsc.load_gather(tmp_v, [zero16, fold])
            nm2 = nm2 + plsc.load_gather(tmp_v, [one16, fold])
            ng2 = ng2 + plsc.load_gather(tmp_v, [two16, fold])
        d = jnp.maximum(nm2, 1e-16) * jnp.maximum(ng2, 1e-16)
        di = plsc.bitcast(d, jnp.int32)
        y = plsc.bitcast(jnp.int32(0x5F3759DF) - (di >> 1), jnp.float32)
        for _ in range(3):
            y = y * (1.5 - 0.5 * d * y * y)
        cos = dot * y
        z = cos * w + bb
        sig = 1.0 / (1.0 + jnp.exp(-z))
        plsc.store_compressed(res_v.at[pl.ds(goff, 16)], sig, mask=mask4)

    fire(0, win0_v, gr0_v, sem0)

    def pair(g2, _):
        g0 = g2 * 2
        fire(g0 + 1, win1_v, gr1_v, sem1)
        drain(g0, win0_v, gr0_v, sem0)
        compute(g0, win0_v, gr0_v)

        @pl.when(g0 + 2 < GROUPS)
        def _():
            fire(g0 + 2, win0_v, gr0_v, sem0)

        drain(g0 + 1, win1_v, gr1_v, sem1)
        compute(g0 + 1, win1_v, gr1_v)
        return 0

    lax.fori_loop(0, GROUPS // 2, pair, 0)

    for c in range(ROWS_PER_W // 128):
        pltpu.async_copy(res_v.at[pl.ds(c * 128, 128)],
                         out_hbm.at[perm_v.at[c]], sem0)
    for c in range(ROWS_PER_W // 128):
        pltpu.make_async_copy(res_v.at[pl.ds(c * 128, 128)],
                              out_hbm.at[pl.ds(0, 128)], sem0).wait()


@jax.jit
def _run(mT, g_table, midx, gidx, perm, wb):
    mesh = plsc.VectorSubcoreMesh(core_axis_name="c", subcore_axis_name="s")
    f = functools.partial(
        pl.kernel,
        mesh=mesh,
        out_type=jax.ShapeDtypeStruct((B,), jnp.float32),
        scratch_types=[
            pltpu.VMEM((ROWS_PER_W + PAD,), jnp.int32),
            pltpu.VMEM((ROWS_PER_W + PAD,), jnp.int32),
            pltpu.VMEM((ROWS_PER_W // 128, 128), jnp.int32),
            pltpu.VMEM((GS, DIM, 128), jnp.float32),
            pltpu.VMEM((GS, DIM, 128), jnp.float32),
            pltpu.VMEM((GS, DIM), jnp.float32),
            pltpu.VMEM((GS, DIM), jnp.float32),
            pltpu.VMEM((ROWS_PER_W + PAD,), jnp.float32),
            pltpu.VMEM((3, 16), jnp.float32),
            pltpu.VMEM((2, 16), jnp.float32),
            pltpu.SemaphoreType.DMA,
            pltpu.SemaphoreType.DMA,
        ],
        compiler_params=pltpu.CompilerParams(needs_layout_passes=False),
    )(_body)
    return f(mT, g_table, midx, gidx, perm, wb)


def kernel(x, m_table, g_table, fc_w, fc_b):
    mT = m_table.T
    midx = x[:, 0].astype(jnp.int32)
    gidx = x[:, 1].astype(jnp.int32)
    order = jnp.argsort(midx).astype(jnp.int32)
    wb = jnp.stack([
        jnp.broadcast_to(fc_w.reshape(()), (16,)),
        jnp.broadcast_to(fc_b.reshape(()), (16,)),
    ]).astype(jnp.float32)
    out = _run(mT, g_table, midx, gidx, order, wb)
    return out.reshape(B, 1)


# speedup vs baseline: 1.1300x; 1.1300x over previous
import functools

import jax
import jax.numpy as jnp
from jax import lax
from jax.experimental import pallas as pl
from jax.experimental.pallas import tpu as pltpu
from jax.experimental.pallas import tpu_sc as plsc

B = 16384
DIM = 64
NW = 32
ROWS_PER_W = B // NW
GS = 7
GROUPS = 74
PAD = 16


def _body(mT_hbm, g_hbm, midx_hbm, gidx_hbm, perm_hbm, wb_hbm, out_hbm,
          midx_v, gidx_v, perm_v, win0_v, win1_v, gr0_v, gr1_v, res_v,
          tmp_v, wb_v, sem0, sem1):
    wid = lax.axis_index("s") * 2 + lax.axis_index("c")
    base = wid * ROWS_PER_W

    pltpu.sync_copy(wb_hbm, wb_v)
    for c in range(ROWS_PER_W // 128):
        pltpu.sync_copy(perm_hbm.at[pl.ds(base + c * 128, 128)],
                        perm_v.at[c])
    for c in range(ROWS_PER_W // 128):
        pltpu.async_copy(midx_hbm.at[perm_v.at[c]],
                         midx_v.at[pl.ds(c * 128, 128)], sem0)
        pltpu.async_copy(gidx_hbm.at[perm_v.at[c]],
                         gidx_v.at[pl.ds(c * 128, 128)], sem1)
    for c in range(ROWS_PER_W // 128):
        pltpu.make_async_copy(midx_hbm.at[pl.ds(0, 128)],
                              midx_v.at[pl.ds(c * 128, 128)], sem0).wait()
        pltpu.make_async_copy(gidx_hbm.at[pl.ds(0, 128)],
                              gidx_v.at[pl.ds(c * 128, 128)], sem1).wait()
    zpad = jnp.zeros((16,), jnp.int32)
    midx_v[pl.ds(ROWS_PER_W, 16)] = zpad
    gidx_v[pl.ds(ROWS_PER_W, 16)] = zpad

    w = wb_v[0, :]
    bb = wb_v[1, :]
    iota = lax.iota(jnp.int32, 16)
    samp16 = jnp.minimum(iota >> 1, GS - 1)
    jhalf = iota & 1
    fold1 = iota ^ 1
    maskst = ((iota & 1) == 0) & (iota < 2 * GS)
    zero16 = jnp.zeros((16,), jnp.int32)
    one16 = jnp.ones((16,), jnp.int32)
    two16 = jnp.full((16,), 2, jnp.int32)

    def windows(g):
        mv = midx_v[pl.ds(g * GS, 16)]
        ws = [mv[k] >> 7 for k in range(GS)]
        news = [True] + [ws[k] != ws[k - 1] for k in range(1, GS)]
        return ws, news

    def fire(g, win_v, gr_v, sem):
        gv = gidx_v[pl.ds(g * GS, 16)]
        ws, news = windows(g)
        for k in range(GS):
            woff = pl.multiple_of(ws[k] * 128, 128)
            if k == 0:
                pltpu.async_copy(mT_hbm.at[:, pl.ds(woff, 128)],
                                 win_v.at[k], sem)
            else:
                @pl.when(news[k])
                def _(woff=woff, k=k):
                    pltpu.async_copy(mT_hbm.at[:, pl.ds(woff, 128)],
                                     win_v.at[k], sem)
            pltpu.async_copy(g_hbm.at[pl.ds(gv[k], 1)],
                             gr_v.at[pl.ds(k, 1)], sem)

    def drain(g, win_v, gr_v, sem):
        _, news = windows(g)
        for k in range(GS):
            if k == 0:
                pltpu.make_async_copy(mT_hbm.at[:, pl.ds(0, 128)],
                                      win_v.at[k], sem).wait()
            else:
                @pl.when(news[k])
                def _(k=k):
                    pltpu.make_async_copy(mT_hbm.at[:, pl.ds(0, 128)],
                                          win_v.at[k], sem).wait()
        pltpu.make_async_copy(g_hbm.at[pl.ds(0, GS)], gr_v, sem).wait()

    def compute(g, win_v, gr_v):
        goff = g * GS
        ws, news = windows(g)
        slots = [jnp.int32(0)]
        for k in range(1, GS):
            slots.append(jnp.where(news[k], jnp.int32(k), slots[k - 1]))
        slotvec = slots[GS - 1]
        for k in range(GS - 2, -1, -1):
            slotvec = jnp.where(samp16 == k, slots[k], slotvec)
        lanevec = plsc.load_gather(midx_v, [goff + samp16]) & 127
        zero = jnp.zeros((16,), jnp.float32)
        dot = zero
        nm2 = zero
        ng2 = zero
        for jb in range(0, DIM, 2):
            jv = jhalf + jb
            mj = plsc.load_gather(win_v, [slotvec, jv, lanevec])
            gj = plsc.load_gather(gr_v, [samp16, jv])
            dot = dot + mj * gj
            nm2 = nm2 + mj * mj
            ng2 = ng2 + gj * gj
        tmp_v[0, :] = dot
        tmp_v[1, :] = nm2
        tmp_v[2, :] = ng2
        dot = dot + plsc.load_gather(tmp_v, [zero16, fold1])
        nm2 = nm2 + plsc.load_gather(tmp_v, [one16, fold1])
        ng2 = ng2 + plsc.load_gather(tmp_v, [two16, fold1])
        d = jnp.maximum(nm2, 1e-16) * jnp.maximum(ng2, 1e-16)
        di = plsc.bitcast(d, jnp.int32)
        y = plsc.bitcast(jnp.int32(0x5F3759DF) - (di >> 1), jnp.float32)
        for _ in range(3):
            y = y * (1.5 - 0.5 * d * y * y)
        cos = dot * y
        z = cos * w + bb
        sig = 1.0 / (1.0 + jnp.exp(-z))
        plsc.store_compressed(res_v.at[pl.ds(goff, 16)], sig, mask=maskst)

    fire(0, win0_v, gr0_v, sem0)

    def pair(g2, _):
        g0 = g2 * 2
        fire(g0 + 1, win1_v, gr1_v, sem1)
        drain(g0, win0_v, gr0_v, sem0)
        compute(g0, win0_v, gr0_v)

        @pl.when(g0 + 2 < GROUPS)
        def _():
            fire(g0 + 2, win0_v, gr0_v, sem0)

        drain(g0 + 1, win1_v, gr1_v, sem1)
        compute(g0 + 1, win1_v, gr1_v)
        return 0

    lax.fori_loop(0, GROUPS // 2, pair, 0)

    for c in range(ROWS_PER_W // 128):
        pltpu.async_copy(res_v.at[pl.ds(c * 128, 128)],
                         out_hbm.at[perm_v.at[c]], sem0)
    for c in range(ROWS_PER_W // 128):
        pltpu.make_async_copy(res_v.at[pl.ds(c * 128, 128)],
                              out_hbm.at[pl.ds(0, 128)], sem0).wait()


@jax.jit
def _run(mT, g_table, midx, gidx, perm, wb):
    mesh = plsc.VectorSubcoreMesh(core_axis_name="c", subcore_axis_name="s")
    f = functools.partial(
        pl.kernel,
        mesh=mesh,
        out_type=jax.ShapeDtypeStruct((B,), jnp.float32),
        scratch_types=[
            pltpu.VMEM((GROUPS * GS + PAD,), jnp.int32),
            pltpu.VMEM((GROUPS * GS + PAD,), jnp.int32),
            pltpu.VMEM((ROWS_PER_W // 128, 128), jnp.int32),
            pltpu.VMEM((GS, DIM, 128), jnp.float32),
            pltpu.VMEM((GS, DIM, 128), jnp.float32),
            pltpu.VMEM((GS, DIM), jnp.float32),
            pltpu.VMEM((GS, DIM), jnp.float32),
            pltpu.VMEM((GROUPS * GS + PAD,), jnp.float32),
            pltpu.VMEM((3, 16), jnp.float32),
            pltpu.VMEM((2, 16), jnp.float32),
            pltpu.SemaphoreType.DMA,
            pltpu.SemaphoreType.DMA,
        ],
        compiler_params=pltpu.CompilerParams(needs_layout_passes=False),
    )(_body)
    return f(mT, g_table, midx, gidx, perm, wb)


def kernel(x, m_table, g_table, fc_w, fc_b):
    mT = m_table.T
    midx = x[:, 0].astype(jnp.int32)
    gidx = x[:, 1].astype(jnp.int32)
    order = jnp.argsort(midx).astype(jnp.int32)
    wb = jnp.stack([
        jnp.broadcast_to(fc_w.reshape(()), (16,)),
        jnp.broadcast_to(fc_b.reshape(()), (16,)),
    ]).astype(jnp.float32)
    out = _run(mT, g_table, midx, gidx, order, wb)
    return out.reshape(B, 1)


# speedup vs baseline: 1.1409x; 1.0096x over previous
import functools

import jax
import jax.numpy as jnp
from jax import lax
from jax.experimental import pallas as pl
from jax.experimental.pallas import tpu as pltpu
from jax.experimental.pallas import tpu_sc as plsc

B = 16384
DIM = 64
NW = 32
ROWS_PER_W = B // NW
GS = 7
GROUPS = 74
PAD = 16


def _body(mT_hbm, g_hbm, midx_hbm, gidx_hbm, perm_hbm, wb_hbm, out_hbm,
          midx_v, gidx_v, perm_v, win0_v, win1_v, gr0_v, gr1_v, res_v,
          tmp_v, wb_v, sem0, sem1):
    wid = lax.axis_index("s") * 2 + lax.axis_index("c")
    base = wid * ROWS_PER_W

    pltpu.sync_copy(wb_hbm, wb_v)
    for c in range(ROWS_PER_W // 128):
        pltpu.sync_copy(perm_hbm.at[pl.ds(base + c * 128, 128)],
                        perm_v.at[c])
    for c in range(ROWS_PER_W // 128):
        pltpu.async_copy(midx_hbm.at[perm_v.at[c]],
                         midx_v.at[pl.ds(c * 128, 128)], sem0)
        pltpu.async_copy(gidx_hbm.at[perm_v.at[c]],
                         gidx_v.at[pl.ds(c * 128, 128)], sem1)
    for c in range(ROWS_PER_W // 128):
        pltpu.make_async_copy(midx_hbm.at[pl.ds(0, 128)],
                              midx_v.at[pl.ds(c * 128, 128)], sem0).wait()
        pltpu.make_async_copy(gidx_hbm.at[pl.ds(0, 128)],
                              gidx_v.at[pl.ds(c * 128, 128)], sem1).wait()
    zpad = jnp.zeros((16,), jnp.int32)
    midx_v[pl.ds(ROWS_PER_W, 16)] = zpad
    gidx_v[pl.ds(ROWS_PER_W, 16)] = zpad

    w = wb_v[0, :]
    bb = wb_v[1, :]
    iota = lax.iota(jnp.int32, 16)
    samp16 = jnp.minimum(iota >> 1, GS - 1)
    jhalf = iota & 1
    fold1 = iota ^ 1
    maskst = ((iota & 1) == 0) & (iota < 2 * GS)
    zero16 = jnp.zeros((16,), jnp.int32)
    one16 = jnp.ones((16,), jnp.int32)
    two16 = jnp.full((16,), 2, jnp.int32)

    def windows(g):
        mv = midx_v[pl.ds(g * GS, 16)]
        ws = [mv[k] >> 7 for k in range(GS)]
        news = [True] + [ws[k] != ws[k - 1] for k in range(1, GS)]
        return ws, news

    def fire(g, win_v, gr_v, sem):
        gv = gidx_v[pl.ds(g * GS, 16)]
        ws, news = windows(g)
        for k in range(GS):
            woff = pl.multiple_of(ws[k] * 128, 128)
            if k == 0:
                pltpu.async_copy(mT_hbm.at[:, pl.ds(woff, 128)],
                                 win_v.at[k], sem)
            else:
                @pl.when(news[k])
                def _(woff=woff, k=k):
                    pltpu.async_copy(mT_hbm.at[:, pl.ds(woff, 128)],
                                     win_v.at[k], sem)
            pltpu.async_copy(g_hbm.at[pl.ds(gv[k], 1)],
                             gr_v.at[pl.ds(k, 1)], sem)

    def drain(g, win_v, gr_v, sem):
        _, news = windows(g)
        for k in range(GS):
            if k == 0:
                pltpu.make_async_copy(mT_hbm.at[:, pl.ds(0, 128)],
                                      win_v.at[k], sem).wait()
            else:
                @pl.when(news[k])
                def _(k=k):
                    pltpu.make_async_copy(mT_hbm.at[:, pl.ds(0, 128)],
                                          win_v.at[k], sem).wait()
        pltpu.make_async_copy(g_hbm.at[pl.ds(0, GS)], gr_v, sem).wait()

    def compute(g, win_v, gr_v):
        goff = g * GS
        ws, news = windows(g)
        slots = [jnp.int32(0)]
        for k in range(1, GS):
            slots.append(jnp.where(news[k], jnp.int32(k), slots[k - 1]))
        slotvec = slots[GS - 1]
        for k in range(GS - 2, -1, -1):
            slotvec = jnp.where(samp16 == k, slots[k], slotvec)
        lanevec = plsc.load_gather(midx_v, [goff + samp16]) & 127
        zero = jnp.zeros((16,), jnp.float32)
        dot = zero
        nm2 = zero
        ng2 = zero
        for jb in range(0, DIM, 2):
            jv = jhalf + jb
            mj = plsc.load_gather(win_v, [slotvec, jv, lanevec])
            gj = plsc.load_gather(gr_v, [samp16, jv])
            dot = dot + mj * gj
            nm2 = nm2 + mj * mj
            ng2 = ng2 + gj * gj
        tmp_v[0, :] = dot
        tmp_v[1, :] = nm2
        tmp_v[2, :] = ng2
        dot = dot + plsc.load_gather(tmp_v, [zero16, fold1])
        nm2 = nm2 + plsc.load_gather(tmp_v, [one16, fold1])
        ng2 = ng2 + plsc.load_gather(tmp_v, [two16, fold1])
        d = jnp.maximum(nm2, 1e-16) * jnp.maximum(ng2, 1e-16)
        di = plsc.bitcast(d, jnp.int32)
        y = plsc.bitcast(jnp.int32(0x5F3759DF) - (di >> 1), jnp.float32)
        for _ in range(3):
            y = y * (1.5 - 0.5 * d * y * y)
        cos = dot * y
        z = cos * w + bb
        sig = 1.0 / (1.0 + jnp.exp(-z))
        plsc.store_compressed(res_v.at[pl.ds(goff, 16)], sig, mask=maskst)

    fire(0, win0_v, gr0_v, sem0)

    def pair(g2, _):
        g0 = g2 * 2
        fire(g0 + 1, win1_v, gr1_v, sem1)
        drain(g0, win0_v, gr0_v, sem0)
        compute(g0, win0_v, gr0_v)

        @pl.when(g0 + 2 < GROUPS)
        def _():
            fire(g0 + 2, win0_v, gr0_v, sem0)

        drain(g0 + 1, win1_v, gr1_v, sem1)
        compute(g0 + 1, win1_v, gr1_v)
        return 0

    lax.fori_loop(0, GROUPS // 2, pair, 0)

    for c in range(ROWS_PER_W // 128):
        pltpu.async_copy(res_v.at[pl.ds(c * 128, 128)],
                         out_hbm.at[perm_v.at[c]], sem0)
    for c in range(ROWS_PER_W // 128):
        pltpu.make_async_copy(res_v.at[pl.ds(c * 128, 128)],
                              out_hbm.at[pl.ds(0, 128)], sem0).wait()


@jax.jit
def _run(mT, g_table, midx, gidx, perm, wb):
    mesh = plsc.VectorSubcoreMesh(core_axis_name="c", subcore_axis_name="s")
    f = functools.partial(
        pl.kernel,
        mesh=mesh,
        out_type=jax.ShapeDtypeStruct((B,), jnp.float32),
        scratch_types=[
            pltpu.VMEM((GROUPS * GS + PAD,), jnp.int32),
            pltpu.VMEM((GROUPS * GS + PAD,), jnp.int32),
            pltpu.VMEM((ROWS_PER_W // 128, 128), jnp.int32),
            pltpu.VMEM((GS, DIM, 128), jnp.float32),
            pltpu.VMEM((GS, DIM, 128), jnp.float32),
            pltpu.VMEM((GS, DIM), jnp.float32),
            pltpu.VMEM((GS, DIM), jnp.float32),
            pltpu.VMEM((GROUPS * GS + PAD,), jnp.float32),
            pltpu.VMEM((3, 16), jnp.float32),
            pltpu.VMEM((2, 16), jnp.float32),
            pltpu.SemaphoreType.DMA,
            pltpu.SemaphoreType.DMA,
        ],
        compiler_params=pltpu.CompilerParams(needs_layout_passes=False),
    )(_body)
    return f(mT, g_table, midx, gidx, perm, wb)


def kernel(x, m_table, g_table, fc_w, fc_b):
    mT = m_table.T
    midx = x[:, 0].astype(jnp.int32)
    gidx = x[:, 1].astype(jnp.int32)
    order = jnp.argsort(midx, stable=False).astype(jnp.int32)
    wb = jnp.stack([
        jnp.broadcast_to(fc_w.reshape(()), (16,)),
        jnp.broadcast_to(fc_b.reshape(()), (16,)),
    ]).astype(jnp.float32)
    out = _run(mT, g_table, midx, gidx, order, wb)
    return out.reshape(B, 1)
